# baseline (speedup 1.0000x reference)
import functools

import jax
import jax.numpy as jnp
from jax import lax
from jax.experimental import pallas as pl
from jax.experimental.pallas import tpu as pltpu
from jax.experimental.pallas import tpu_sc as plsc

S, N, E = 4096, 4, 1024
B = S * N
NW = 32
RPW = B // NW
CH = 32
NCHUNK = RPW // CH
LANES = 16
NCOL = E // LANES


def _make_sc_call():
    mesh = plsc.VectorSubcoreMesh(core_axis_name="c", subcore_axis_name="s")

    @functools.partial(
        pl.kernel,
        mesh=mesh,
        out_type=jax.ShapeDtypeStruct((B, E), jnp.float32),
        scratch_types=[
            pltpu.VMEM((CH, E), jnp.float32),
            pltpu.VMEM((2, E), jnp.float32),
            pltpu.VMEM((RPW,), jnp.int32),
            pltpu.VMEM((CH * LANES,), jnp.float32),
        ],
    )
    def sc_call(seq_hbm, tok_hbm, table_hbm, out_hbm, buf, table_v, tok_v, f_v):
        wid = lax.axis_index("s") * 2 + lax.axis_index("c")
        base = wid * RPW
        pltpu.sync_copy(tok_hbm.at[pl.ds(base, RPW)], tok_v)
        pltpu.sync_copy(table_hbm, table_v)

        def col_body(c, _):
            o = c * LANES
            t0c = table_v[0, pl.ds(o, LANES)]
            dc = table_v[1, pl.ds(o, LANES)] - t0c
            for r in range(CH):
                fr = f_v[pl.ds(r * LANES, LANES)]
                s = buf[r, pl.ds(o, LANES)]
                buf[r, pl.ds(o, LANES)] = s + t0c + fr * dc
            return 0

        def chunk_body(i, _):
            row0 = i * CH
            pltpu.sync_copy(seq_hbm.at[pl.ds(base + row0, CH)], buf)
            dnums = lax.GatherDimensionNumbers(
                offset_dims=(), collapsed_slice_dims=(0,),
                start_index_map=(0,))
            for g in range(CH // LANES):
                tv = tok_v[pl.ds(row0 + g * LANES, LANES)].astype(jnp.float32)
                for l in range(LANES):
                    idx = jnp.full((LANES, 1), l, jnp.int32)
                    f_v[pl.ds((g * LANES + l) * LANES, LANES)] = lax.gather(
                        tv, idx, dnums, slice_sizes=(1,),
                        mode=lax.GatherScatterMode.PROMISE_IN_BOUNDS)
            lax.fori_loop(0, NCOL, col_body, 0)
            pltpu.sync_copy(buf, out_hbm.at[pl.ds(base + row0, CH)])
            return 0

        lax.fori_loop(0, NCHUNK, chunk_body, 0)

    return sc_call


_sc_call = _make_sc_call()


@jax.jit
def kernel(seq_input, token_type_input, token_type_embeddings):
    seq = seq_input.reshape(B, E)
    tok = token_type_input.reshape(B).astype(jnp.int32)
    out = _sc_call(seq, tok, token_type_embeddings)
    return out.reshape(S, N, E)


# speedup vs baseline: 1.1345x; 1.1345x over previous
import functools

import jax
import jax.numpy as jnp
from jax import lax
from jax.experimental import pallas as pl
from jax.experimental.pallas import tpu as pltpu
from jax.experimental.pallas import tpu_sc as plsc

S, N, E = 4096, 4, 1024
B = S * N
NW = 32
RPW = B // NW
CH = 32
NCHUNK = RPW // CH
LANES = 16
NCOL = E // LANES
NBUF = 3


def _make_sc_call():
    mesh = plsc.VectorSubcoreMesh(core_axis_name="c", subcore_axis_name="s")

    @functools.partial(
        pl.kernel,
        mesh=mesh,
        out_type=jax.ShapeDtypeStruct((B, E), jnp.float32),
        scratch_types=[
            pltpu.VMEM((CH, E), jnp.float32),
            pltpu.VMEM((CH, E), jnp.float32),
            pltpu.VMEM((CH, E), jnp.float32),
            pltpu.VMEM((2, E), jnp.float32),
            pltpu.VMEM((RPW,), jnp.int32),
            pltpu.VMEM((RPW * LANES,), jnp.float32),
            pltpu.SemaphoreType.DMA((NBUF,)),
            pltpu.SemaphoreType.DMA((NBUF,)),
        ],
    )
    def sc_call(seq_hbm, tok_hbm, table_hbm, out_hbm,
                buf0, buf1, buf2, table_v, tok_v, f_v, in_sem, out_sem):
        bufs = (buf0, buf1, buf2)
        wid = lax.axis_index("s") * 2 + lax.axis_index("c")
        base = wid * RPW

        in_copies = [None] * NBUF
        in_copies[0] = pltpu.async_copy(
            seq_hbm.at[pl.ds(base, CH)], bufs[0], in_sem.at[0])
        in_copies[1] = pltpu.async_copy(
            seq_hbm.at[pl.ds(base + CH, CH)], bufs[1], in_sem.at[1])

        pltpu.sync_copy(tok_hbm.at[pl.ds(base, RPW)], tok_v)
        pltpu.sync_copy(table_hbm, table_v)

        dnums = lax.GatherDimensionNumbers(
            offset_dims=(), collapsed_slice_dims=(0,), start_index_map=(0,))

        def bcast_body(g, _):
            tv = tok_v[pl.ds(g * LANES, LANES)].astype(jnp.float32)
            for l in range(LANES):
                idx = jnp.full((LANES, 1), l, jnp.int32)
                f_v[pl.ds((g * LANES + l) * LANES, LANES)] = lax.gather(
                    tv, idx, dnums, slice_sizes=(1,),
                    mode=lax.GatherScatterMode.PROMISE_IN_BOUNDS)
            return 0

        lax.fori_loop(0, RPW // LANES, bcast_body, 0)

        out_copies = [None] * NBUF

        for i in range(NCHUNK):
            b = i % NBUF
            buf = bufs[b]
            row0 = i * CH
            in_copies[b].wait()

            def col_body(c, _, buf=buf, row0=row0):
                o = c * LANES
                t0c = table_v[0, pl.ds(o, LANES)]
                dc = table_v[1, pl.ds(o, LANES)] - t0c
                for r in range(CH):
                    fr = f_v[pl.ds((row0 + r) * LANES, LANES)]
                    s = buf[r, pl.ds(o, LANES)]
                    buf[r, pl.ds(o, LANES)] = s + t0c + fr * dc
                return 0

            lax.fori_loop(0, NCOL, col_body, 0)

            out_copies[b] = pltpu.async_copy(
                buf, out_hbm.at[pl.ds(base + row0, CH)], out_sem.at[b])

            nxt = i + 2
            if nxt < NCHUNK:
                b2 = nxt % NBUF
                if out_copies[b2] is not None:
                    out_copies[b2].wait()
                in_copies[b2] = pltpu.async_copy(
                    seq_hbm.at[pl.ds(base + nxt * CH, CH)], bufs[b2],
                    in_sem.at[b2])

        for b in range(NBUF):
            if out_copies[b] is not None:
                out_copies[b].wait()

    return sc_call


_sc_call = _make_sc_call()


@jax.jit
def kernel(seq_input, token_type_input, token_type_embeddings):
    seq = seq_input.reshape(B, E)
    tok = token_type_input.reshape(B).astype(jnp.int32)
    out = _sc_call(seq, tok, token_type_embeddings)
    return out.reshape(S, N, E)


# speedup vs baseline: 1.8250x; 1.6086x over previous
import functools

import jax
import jax.numpy as jnp
from jax import lax
from jax.experimental import pallas as pl
from jax.experimental.pallas import tpu as pltpu
from jax.experimental.pallas import tpu_sc as plsc

S, N, E = 4096, 4, 1024
B = S * N
NW = 32
RPW = B // NW
CH = 16
NCHUNK = RPW // CH
LANES = 16
NGRP = E // LANES
GPB = 16
NCB = NGRP // GPB
NBUF = 4


def _make_sc_call():
    mesh = plsc.VectorSubcoreMesh(core_axis_name="c", subcore_axis_name="s")

    @functools.partial(
        pl.kernel,
        mesh=mesh,
        out_type=jax.ShapeDtypeStruct((B, E), jnp.float32),
        scratch_types=[
            pltpu.VMEM((CH, E), jnp.float32),
            pltpu.VMEM((CH, E), jnp.float32),
            pltpu.VMEM((CH, E), jnp.float32),
            pltpu.VMEM((CH, E), jnp.float32),
            pltpu.VMEM((2, E), jnp.float32),
            pltpu.VMEM((RPW,), jnp.int32),
            pltpu.VMEM((RPW * LANES,), jnp.float32),
            pltpu.SemaphoreType.DMA((NBUF,)),
            pltpu.SemaphoreType.DMA((NBUF,)),
        ],
    )
    def sc_call(seq_hbm, tok_hbm, table_hbm, out_hbm,
                buf0, buf1, buf2, buf3, table_v, tok_v, f_v,
                in_sem, out_sem):
        bufs = (buf0, buf1, buf2, buf3)
        wid = lax.axis_index("s") * 2 + lax.axis_index("c")
        base = wid * RPW

        for b in range(NBUF - 1):
            pltpu.async_copy(
                seq_hbm.at[pl.ds(base + b * CH, CH)], bufs[b], in_sem.at[b])

        pltpu.sync_copy(tok_hbm.at[pl.ds(base, RPW)], tok_v)
        pltpu.sync_copy(table_hbm, table_v)

        dnums = lax.GatherDimensionNumbers(
            offset_dims=(), collapsed_slice_dims=(0,), start_index_map=(0,))

        def bcast_body(g, _):
            tv = tok_v[pl.ds(g * LANES, LANES)].astype(jnp.float32)
            for l in range(LANES):
                idx = jnp.full((LANES, 1), l, jnp.int32)
                f_v[pl.ds((g * LANES + l) * LANES, LANES)] = lax.gather(
                    tv, idx, dnums, slice_sizes=(1,),
                    mode=lax.GatherScatterMode.PROMISE_IN_BOUNDS)
            return 0

        lax.fori_loop(0, RPW // LANES, bcast_body, 0)

        def wait_in(b):
            pltpu.make_async_copy(
                seq_hbm.at[pl.ds(0, CH)], bufs[b], in_sem.at[b]).wait()

        def wait_out(b):
            pltpu.make_async_copy(
                bufs[b], out_hbm.at[pl.ds(0, CH)], out_sem.at[b]).wait()

        def super_step(g, _):
            for b in range(NBUF):
                i = g * NBUF + b
                row0 = i * CH
                wait_in(b)
                buf = bufs[b]

                for cb in range(NCB):
                    o0 = cb * GPB * LANES
                    t0s = [table_v[0, pl.ds(o0 + k * LANES, LANES)]
                           for k in range(GPB)]
                    dcs = [table_v[1, pl.ds(o0 + k * LANES, LANES)] - t0s[k]
                           for k in range(GPB)]

                    def row_body(r, _, buf=buf, row0=row0, o0=o0,
                                 t0s=t0s, dcs=dcs):
                        fr = f_v[pl.ds((row0 + r) * LANES, LANES)]
                        for k in range(GPB):
                            o = o0 + k * LANES
                            s = buf[r, pl.ds(o, LANES)]
                            buf[r, pl.ds(o, LANES)] = s + t0s[k] + fr * dcs[k]
                        return 0

                    lax.fori_loop(0, CH, row_body, 0)

                pltpu.async_copy(
                    buf, out_hbm.at[pl.ds(base + row0, CH)], out_sem.at[b])

                j = i + NBUF - 1
                bj = (b + NBUF - 1) % NBUF

                @pl.when(jnp.logical_and(j >= NBUF, j < NCHUNK))
                def _():
                    wait_out(bj)

                @pl.when(j < NCHUNK)
                def _():
                    pltpu.async_copy(
                        seq_hbm.at[pl.ds(base + j * CH, CH)], bufs[bj],
                        in_sem.at[bj])

            return 0

        lax.fori_loop(0, NCHUNK // NBUF, super_step, 0)

        for b in range(NBUF):
            wait_out(b)

    return sc_call


_sc_call = _make_sc_call()


@jax.jit
def kernel(seq_input, token_type_input, token_type_embeddings):
    seq = seq_input.reshape(B, E)
    tok = token_type_input.reshape(B).astype(jnp.int32)
    out = _sc_call(seq, tok, token_type_embeddings)
    return out.reshape(S, N, E)


# speedup vs baseline: 5.8327x; 3.1961x over previous
import functools

import jax
import jax.numpy as jnp
from jax import lax
from jax.experimental import pallas as pl
from jax.experimental.pallas import tpu as pltpu
from jax.experimental.pallas import tpu_sc as plsc

S, N, E = 4096, 4, 1024
B = S * N
NW = 32
RPW = B // NW
CH = 16
NCHUNK = RPW // CH
LANES = 16
NGRP = E // LANES
GPB = 16
NCB = NGRP // GPB
NBUF = 4


def _make_sc_call():
    mesh = plsc.VectorSubcoreMesh(core_axis_name="c", subcore_axis_name="s")

    @functools.partial(
        pl.kernel,
        mesh=mesh,
        out_type=jax.ShapeDtypeStruct((S, N, E), jnp.float32),
        scratch_types=[
            pltpu.VMEM((CH // N, N, E), jnp.float32),
            pltpu.VMEM((CH // N, N, E), jnp.float32),
            pltpu.VMEM((CH // N, N, E), jnp.float32),
            pltpu.VMEM((CH // N, N, E), jnp.float32),
            pltpu.VMEM((2, E), jnp.float32),
            pltpu.VMEM((RPW,), jnp.int32),
            pltpu.VMEM((RPW * LANES,), jnp.float32),
            pltpu.SemaphoreType.DMA((NBUF,)),
            pltpu.SemaphoreType.DMA((NBUF,)),
        ],
    )
    def sc_call(seq_hbm, tok_hbm, table_hbm, out_hbm,
                buf0, buf1, buf2, buf3, table_v, tok_v, f_v,
                in_sem, out_sem):
        bufs = (buf0, buf1, buf2, buf3)
        wid = lax.axis_index("s") * 2 + lax.axis_index("c")
        base = wid * RPW
        sbase = wid * (RPW // N)
        SCH = CH // N

        for b in range(NBUF - 1):
            pltpu.async_copy(
                seq_hbm.at[pl.ds(sbase + b * SCH, SCH)], bufs[b],
                in_sem.at[b])

        pltpu.sync_copy(tok_hbm.at[pl.ds(base, RPW)], tok_v)
        pltpu.sync_copy(table_hbm, table_v)

        dnums = lax.GatherDimensionNumbers(
            offset_dims=(), collapsed_slice_dims=(0,), start_index_map=(0,))

        def bcast_body(g, _):
            tv = tok_v[pl.ds(g * LANES, LANES)].astype(jnp.float32)
            for l in range(LANES):
                idx = jnp.full((LANES, 1), l, jnp.int32)
                f_v[pl.ds((g * LANES + l) * LANES, LANES)] = lax.gather(
                    tv, idx, dnums, slice_sizes=(1,),
                    mode=lax.GatherScatterMode.PROMISE_IN_BOUNDS)
            return 0

        lax.fori_loop(0, RPW // LANES, bcast_body, 0)

        def wait_in(b):
            pltpu.make_async_copy(
                seq_hbm.at[pl.ds(0, SCH)], bufs[b], in_sem.at[b]).wait()

        def wait_out(b):
            pltpu.make_async_copy(
                bufs[b], out_hbm.at[pl.ds(0, SCH)], out_sem.at[b]).wait()

        def super_step(g, _):
            for b in range(NBUF):
                i = g * NBUF + b
                row0 = i * CH
                wait_in(b)
                buf = bufs[b]

                for cb in range(NCB):
                    o0 = cb * GPB * LANES
                    t0s = [table_v[0, pl.ds(o0 + k * LANES, LANES)]
                           for k in range(GPB)]
                    dcs = [table_v[1, pl.ds(o0 + k * LANES, LANES)] - t0s[k]
                           for k in range(GPB)]

                    def row_body(r, _, buf=buf, row0=row0, o0=o0,
                                 t0s=t0s, dcs=dcs):
                        fr = f_v[pl.ds((row0 + r) * LANES, LANES)]
                        a = r // N
                        n = r % N
                        for k in range(GPB):
                            o = o0 + k * LANES
                            s = buf[a, n, pl.ds(o, LANES)]
                            buf[a, n, pl.ds(o, LANES)] = (
                                s + t0s[k] + fr * dcs[k])
                        return 0

                    lax.fori_loop(0, CH, row_body, 0)

                pltpu.async_copy(
                    buf, out_hbm.at[pl.ds(sbase + i * SCH, SCH)],
                    out_sem.at[b])

                j = i + NBUF - 1
                bj = (b + NBUF - 1) % NBUF

                @pl.when(jnp.logical_and(j >= NBUF, j < NCHUNK))
                def _():
                    wait_out(bj)

                @pl.when(j < NCHUNK)
                def _():
                    pltpu.async_copy(
                        seq_hbm.at[pl.ds(sbase + j * SCH, SCH)], bufs[bj],
                        in_sem.at[bj])

            return 0

        lax.fori_loop(0, NCHUNK // NBUF, super_step, 0)

        for b in range(NBUF):
            wait_out(b)

    return sc_call


_sc_call = _make_sc_call()


@jax.jit
def kernel(seq_input, token_type_input, token_type_embeddings):
    tok = token_type_input.reshape(B).astype(jnp.int32)
    return _sc_call(seq_input, tok, token_type_embeddings)
